# baseline (device time: 93048 ns/iter reference)
import jax
import jax.numpy as jnp
from jax import lax
from jax.experimental import pallas as pl
from jax.experimental.pallas import tpu as pltpu

N_DEV = 4
SQ = 1024
SKV = 1024
HQ_LOCAL = 8
DH = 128
D_LOCAL = HQ_LOCAL * DH
D_MODEL = 1024
BLK = 64
SCALE = 0.08838834764831843
NEG = -1e9

RC = SQ // N_DEV
CH = D_MODEL // 2


def kernel(x, Wq, K_ext, V_ext, Wo):
    x2 = x[0]
    k2 = K_ext.reshape(SKV, HQ_LOCAL * DH)
    v2 = V_ext.reshape(SKV, HQ_LOCAL * DH)

    def body(x_ref, wq_ref, k_ref, v_ref, wo_ref, out_ref,
             ctx_ref, acw_ref, accw_ref, rcw_ref, rccw_ref,
             send_cw, recv_cw, send_ccw, recv_ccw):
        my = lax.axis_index("i")
        left = lax.rem(my + N_DEV - 1, N_DEV)
        right = lax.rem(my + 1, N_DEV)

        barrier_sem = pltpu.get_barrier_semaphore()
        for nbr in (left, right):
            pl.semaphore_signal(
                barrier_sem, inc=1,
                device_id=(nbr,), device_id_type=pl.DeviceIdType.MESH,
            )
        pl.semaphore_wait(barrier_sem, 2)

        q_all = jnp.dot(x_ref[...], wq_ref[:, pl.ds(my * D_LOCAL, D_LOCAL)],
                        preferred_element_type=jnp.float32)

        qb = lax.broadcasted_iota(jnp.int32, (SQ, SKV), 0) // BLK
        kb = lax.broadcasted_iota(jnp.int32, (SQ, SKV), 1) // BLK
        mask = kb <= qb

        for h in range(HQ_LOCAL):
            sl = slice(h * DH, (h + 1) * DH)
            q = q_all[:, sl]
            k = k_ref[:, sl]
            v = v_ref[:, sl]
            s = lax.dot_general(
                q, k, (((1,), (1,)), ((), ())),
                preferred_element_type=jnp.float32,
            ) * SCALE
            s = jnp.where(mask, s, NEG)
            m = jnp.max(s, axis=1, keepdims=True)
            p = jnp.exp(s - m)
            denom = jnp.sum(p, axis=1, keepdims=True)
            p = p / denom
            ctx_ref[:, sl] = jnp.dot(p, v, preferred_element_type=jnp.float32)

        wo_l = wo_ref[pl.ds(my * D_LOCAL, D_LOCAL), :]
        for c in range(N_DEV):
            pc = jnp.dot(ctx_ref[c * RC:(c + 1) * RC, :], wo_l,
                         preferred_element_type=jnp.float32)
            acw_ref[c] = pc[:, :CH]
            accw_ref[c] = pc[:, CH:]

        def cmod(k):
            return lax.rem(k + 4 * N_DEV, N_DEV)

        for s in range(N_DEV - 1):
            cw = pltpu.make_async_remote_copy(
                src_ref=acw_ref.at[cmod(my - s)],
                dst_ref=rcw_ref.at[s],
                send_sem=send_cw.at[s],
                recv_sem=recv_cw.at[s],
                device_id=(right,),
                device_id_type=pl.DeviceIdType.MESH,
            )
            ccw = pltpu.make_async_remote_copy(
                src_ref=accw_ref.at[cmod(my + s)],
                dst_ref=rccw_ref.at[s],
                send_sem=send_ccw.at[s],
                recv_sem=recv_ccw.at[s],
                device_id=(left,),
                device_id_type=pl.DeviceIdType.MESH,
            )
            cw.start()
            ccw.start()
            cw.wait()
            ccw.wait()
            icw = cmod(my - s - 1)
            iccw = cmod(my + s + 1)
            acw_ref[icw] = acw_ref[icw] + rcw_ref[s]
            accw_ref[iccw] = accw_ref[iccw] + rccw_ref[s]

        own_cw = cmod(my + 1)
        own_ccw = cmod(my - 1)
        out_ref[pl.ds(own_cw * RC, RC), :CH] = acw_ref[own_cw]
        out_ref[pl.ds(own_ccw * RC, RC), CH:] = accw_ref[own_ccw]

        for s in range(N_DEV - 1):
            c_cw = cmod(my + 1 - s)
            c_ccw = cmod(my - 1 + s)
            src_cw = (acw_ref.at[c_cw] if s == 0
                      else out_ref.at[pl.ds(c_cw * RC, RC), pl.ds(0, CH)])
            src_ccw = (accw_ref.at[c_ccw] if s == 0
                       else out_ref.at[pl.ds(c_ccw * RC, RC), pl.ds(CH, CH)])
            cw = pltpu.make_async_remote_copy(
                src_ref=src_cw,
                dst_ref=out_ref.at[pl.ds(c_cw * RC, RC), pl.ds(0, CH)],
                send_sem=send_cw.at[3 + s],
                recv_sem=recv_cw.at[3 + s],
                device_id=(right,),
                device_id_type=pl.DeviceIdType.MESH,
            )
            ccw = pltpu.make_async_remote_copy(
                src_ref=src_ccw,
                dst_ref=out_ref.at[pl.ds(c_ccw * RC, RC), pl.ds(CH, CH)],
                send_sem=send_ccw.at[3 + s],
                recv_sem=recv_ccw.at[3 + s],
                device_id=(left,),
                device_id_type=pl.DeviceIdType.MESH,
            )
            cw.start()
            ccw.start()
            cw.wait()
            ccw.wait()

    out = pl.pallas_call(
        body,
        out_shape=jax.ShapeDtypeStruct((SQ, D_MODEL), jnp.float32),
        in_specs=[pl.BlockSpec(memory_space=pltpu.VMEM)] * 5,
        out_specs=pl.BlockSpec(memory_space=pltpu.VMEM),
        scratch_shapes=[
            pltpu.VMEM((SQ, D_LOCAL), jnp.float32),
            pltpu.VMEM((N_DEV, RC, CH), jnp.float32),
            pltpu.VMEM((N_DEV, RC, CH), jnp.float32),
            pltpu.VMEM((N_DEV - 1, RC, CH), jnp.float32),
            pltpu.VMEM((N_DEV - 1, RC, CH), jnp.float32),
            pltpu.SemaphoreType.DMA((6,)),
            pltpu.SemaphoreType.DMA((6,)),
            pltpu.SemaphoreType.DMA((6,)),
            pltpu.SemaphoreType.DMA((6,)),
        ],
        compiler_params=pltpu.CompilerParams(
            collective_id=0,
            vmem_limit_bytes=100 * 1024 * 1024,
        ),
    )(x2, Wq, k2, v2, Wo)
    return out[None]


# device time: 78464 ns/iter; 1.1859x vs baseline; 1.1859x over previous
import jax
import jax.numpy as jnp
from jax import lax
from jax.experimental import pallas as pl
from jax.experimental.pallas import tpu as pltpu

N_DEV = 4
SQ = 1024
SKV = 1024
HQ_LOCAL = 8
DH = 128
D_LOCAL = HQ_LOCAL * DH
D_MODEL = 1024
BLK = 64
SCALE = 0.08838834764831843
NEG = -1e9

RC = SQ // N_DEV
CH = D_MODEL // 2


def kernel(x, Wq, K_ext, V_ext, Wo):
    x2 = x[0]
    k2 = K_ext.reshape(SKV, HQ_LOCAL * DH)
    v2 = V_ext.reshape(SKV, HQ_LOCAL * DH)

    def body(x_ref, wq_ref, k_ref, v_ref, wo_ref, out_ref,
             ctx_ref, acw_ref, accw_ref, rcw_ref, rccw_ref,
             wq_l_ref, wo_l_ref, dma_sems,
             send_cw, recv_cw, send_ccw, recv_ccw):
        my = lax.axis_index("i")
        left = lax.rem(my + N_DEV - 1, N_DEV)
        right = lax.rem(my + 1, N_DEV)

        def cmod(k):
            return lax.rem(k + 4 * N_DEV, N_DEV)

        cp_wq = pltpu.make_async_copy(
            wq_ref.at[:, pl.ds(my * D_LOCAL, D_LOCAL)], wq_l_ref,
            dma_sems.at[0])
        cp_wo = pltpu.make_async_copy(
            wo_ref.at[pl.ds(my * D_LOCAL, D_LOCAL), :], wo_l_ref,
            dma_sems.at[1])
        cp_wq.start()
        cp_wo.start()

        barrier_sem = pltpu.get_barrier_semaphore()
        for nbr in (left, right):
            pl.semaphore_signal(
                barrier_sem, inc=1,
                device_id=(nbr,), device_id_type=pl.DeviceIdType.MESH,
            )
        pl.semaphore_wait(barrier_sem, 2)

        cp_wq.wait()
        cp_wo.wait()
        wq_l = wq_l_ref[...]
        wo_l = wo_l_ref[...]
        kb = lax.broadcasted_iota(jnp.int32, (RC, SKV), 1) // BLK
        iota_r = lax.broadcasted_iota(jnp.int32, (RC, SKV), 0)

        def compute_chunk(c):
            r0 = c * RC
            q_c = jnp.dot(x_ref[pl.ds(r0, RC), :], wq_l,
                          preferred_element_type=jnp.float32)
            qb = (iota_r + r0) // BLK
            mask = kb <= qb
            for h in range(HQ_LOCAL):
                sl = slice(h * DH, (h + 1) * DH)
                s = lax.dot_general(
                    q_c[:, sl], k_ref[:, sl], (((1,), (1,)), ((), ())),
                    preferred_element_type=jnp.float32,
                ) * SCALE
                s = jnp.where(mask, s, NEG)
                m = jnp.max(s, axis=1, keepdims=True)
                p = jnp.exp(s - m)
                denom = jnp.sum(p, axis=1, keepdims=True)
                p = p / denom
                ctx_ref[pl.ds(r0, RC), sl] = jnp.dot(
                    p, v_ref[:, sl], preferred_element_type=jnp.float32)
            pc = jnp.dot(ctx_ref[pl.ds(r0, RC), :], wo_l,
                         preferred_element_type=jnp.float32)
            acw_ref[c] = pc[:, :CH]
            accw_ref[c] = pc[:, CH:]

        def rs_start(s):
            cw = pltpu.make_async_remote_copy(
                src_ref=acw_ref.at[cmod(my - s)],
                dst_ref=rcw_ref.at[s],
                send_sem=send_cw.at[s],
                recv_sem=recv_cw.at[s],
                device_id=(right,),
                device_id_type=pl.DeviceIdType.MESH,
            )
            ccw = pltpu.make_async_remote_copy(
                src_ref=accw_ref.at[cmod(my + s)],
                dst_ref=rccw_ref.at[s],
                send_sem=send_ccw.at[s],
                recv_sem=recv_ccw.at[s],
                device_id=(left,),
                device_id_type=pl.DeviceIdType.MESH,
            )
            cw.start()
            ccw.start()
            return cw, ccw

        def rs_wait_add(s, cw, ccw):
            cw.wait()
            ccw.wait()
            icw = cmod(my - s - 1)
            iccw = cmod(my + s + 1)
            acw_ref[icw] = acw_ref[icw] + rcw_ref[s]
            accw_ref[iccw] = accw_ref[iccw] + rccw_ref[s]

        compute_chunk(my)
        rs0 = rs_start(0)
        compute_chunk(cmod(my - 1))
        compute_chunk(cmod(my + 1))
        rs_wait_add(0, *rs0)
        rs1 = rs_start(1)
        compute_chunk(cmod(my + 2))
        rs_wait_add(1, *rs1)
        rs2 = rs_start(2)
        rs_wait_add(2, *rs2)

        own_cw = cmod(my + 1)
        own_ccw = cmod(my - 1)
        out_ref[pl.ds(own_cw * RC, RC), :CH] = acw_ref[own_cw]
        out_ref[pl.ds(own_ccw * RC, RC), CH:] = accw_ref[own_ccw]

        for s in range(N_DEV - 1):
            c_cw = cmod(my + 1 - s)
            c_ccw = cmod(my - 1 + s)
            src_cw = (acw_ref.at[c_cw] if s == 0
                      else out_ref.at[pl.ds(c_cw * RC, RC), pl.ds(0, CH)])
            src_ccw = (accw_ref.at[c_ccw] if s == 0
                       else out_ref.at[pl.ds(c_ccw * RC, RC), pl.ds(CH, CH)])
            cw = pltpu.make_async_remote_copy(
                src_ref=src_cw,
                dst_ref=out_ref.at[pl.ds(c_cw * RC, RC), pl.ds(0, CH)],
                send_sem=send_cw.at[3 + s],
                recv_sem=recv_cw.at[3 + s],
                device_id=(right,),
                device_id_type=pl.DeviceIdType.MESH,
            )
            ccw = pltpu.make_async_remote_copy(
                src_ref=src_ccw,
                dst_ref=out_ref.at[pl.ds(c_ccw * RC, RC), pl.ds(CH, CH)],
                send_sem=send_ccw.at[3 + s],
                recv_sem=recv_ccw.at[3 + s],
                device_id=(left,),
                device_id_type=pl.DeviceIdType.MESH,
            )
            cw.start()
            ccw.start()
            cw.wait()
            ccw.wait()

    out = pl.pallas_call(
        body,
        out_shape=jax.ShapeDtypeStruct((SQ, D_MODEL), jnp.float32),
        in_specs=[
            pl.BlockSpec(memory_space=pltpu.VMEM),
            pl.BlockSpec(memory_space=pl.ANY),
            pl.BlockSpec(memory_space=pltpu.VMEM),
            pl.BlockSpec(memory_space=pltpu.VMEM),
            pl.BlockSpec(memory_space=pl.ANY),
        ],
        out_specs=pl.BlockSpec(memory_space=pltpu.VMEM),
        scratch_shapes=[
            pltpu.VMEM((SQ, D_LOCAL), jnp.float32),
            pltpu.VMEM((N_DEV, RC, CH), jnp.float32),
            pltpu.VMEM((N_DEV, RC, CH), jnp.float32),
            pltpu.VMEM((N_DEV - 1, RC, CH), jnp.float32),
            pltpu.VMEM((N_DEV - 1, RC, CH), jnp.float32),
            pltpu.VMEM((D_MODEL, D_LOCAL), jnp.float32),
            pltpu.VMEM((D_LOCAL, D_MODEL), jnp.float32),
            pltpu.SemaphoreType.DMA((2,)),
            pltpu.SemaphoreType.DMA((6,)),
            pltpu.SemaphoreType.DMA((6,)),
            pltpu.SemaphoreType.DMA((6,)),
            pltpu.SemaphoreType.DMA((6,)),
        ],
        compiler_params=pltpu.CompilerParams(
            collective_id=0,
            vmem_limit_bytes=100 * 1024 * 1024,
        ),
    )(x2, Wq, k2, v2, Wo)
    return out[None]


# device time: 70784 ns/iter; 1.3145x vs baseline; 1.1085x over previous
import jax
import jax.numpy as jnp
from jax import lax
from jax.experimental import pallas as pl
from jax.experimental.pallas import tpu as pltpu

N_DEV = 4
SQ = 1024
SKV = 1024
HQ_LOCAL = 8
DH = 128
D_LOCAL = HQ_LOCAL * DH
D_MODEL = 1024
BLK = 64
SCALE = 0.08838834764831843
NEG = -1e9

RC = SQ // N_DEV
CH = D_MODEL // 2


def kernel(x, Wq, K_ext, V_ext, Wo):
    x2 = x[0].astype(jnp.bfloat16)
    k2 = K_ext.reshape(SKV, HQ_LOCAL * DH).astype(jnp.bfloat16)
    v2 = V_ext.reshape(SKV, HQ_LOCAL * DH).astype(jnp.bfloat16)

    def body(x_ref, wq_ref, k_ref, v_ref, wo_ref, out_ref,
             ctx_ref, acw_ref, accw_ref, rcw_ref, rccw_ref,
             wq_l_ref, wo_l_ref, dma_sems,
             send_cw, recv_cw, send_ccw, recv_ccw):
        my = lax.axis_index("i")
        left = lax.rem(my + N_DEV - 1, N_DEV)
        right = lax.rem(my + 1, N_DEV)

        def cmod(k):
            return lax.rem(k + 4 * N_DEV, N_DEV)

        cp_wq = pltpu.make_async_copy(
            wq_ref.at[:, pl.ds(my * D_LOCAL, D_LOCAL)], wq_l_ref,
            dma_sems.at[0])
        cp_wo = pltpu.make_async_copy(
            wo_ref.at[pl.ds(my * D_LOCAL, D_LOCAL), :], wo_l_ref,
            dma_sems.at[1])
        cp_wq.start()
        cp_wo.start()

        barrier_sem = pltpu.get_barrier_semaphore()
        for nbr in (left, right):
            pl.semaphore_signal(
                barrier_sem, inc=1,
                device_id=(nbr,), device_id_type=pl.DeviceIdType.MESH,
            )
        pl.semaphore_wait(barrier_sem, 2)

        cp_wq.wait()
        cp_wo.wait()
        wq_l = wq_l_ref[...].astype(jnp.bfloat16)
        wo_l = wo_l_ref[...].astype(jnp.bfloat16)
        kb = lax.broadcasted_iota(jnp.int32, (RC, SKV), 1) // BLK
        iota_r = lax.broadcasted_iota(jnp.int32, (RC, SKV), 0)

        def compute_chunk(c):
            r0 = c * RC
            q_c = jnp.dot(x_ref[pl.ds(r0, RC), :], wq_l,
                          preferred_element_type=jnp.float32
                          ).astype(jnp.bfloat16)
            qb = (iota_r + r0) // BLK
            mask = kb <= qb
            for h in range(HQ_LOCAL):
                sl = slice(h * DH, (h + 1) * DH)
                s = lax.dot_general(
                    q_c[:, sl], k_ref[:, sl], (((1,), (1,)), ((), ())),
                    preferred_element_type=jnp.float32,
                ) * SCALE
                s = jnp.where(mask, s, NEG)
                m = jnp.max(s, axis=1, keepdims=True)
                p = jnp.exp(s - m)
                denom = jnp.sum(p, axis=1, keepdims=True)
                p = (p / denom).astype(jnp.bfloat16)
                ctx_ref[pl.ds(r0, RC), sl] = jnp.dot(
                    p, v_ref[:, sl], preferred_element_type=jnp.float32
                ).astype(jnp.bfloat16)
            pc = jnp.dot(ctx_ref[pl.ds(r0, RC), :], wo_l,
                         preferred_element_type=jnp.float32)
            acw_ref[c] = pc[:, :CH].astype(jnp.bfloat16)
            accw_ref[c] = pc[:, CH:].astype(jnp.bfloat16)

        def rs_start(s):
            cw = pltpu.make_async_remote_copy(
                src_ref=acw_ref.at[cmod(my - s)],
                dst_ref=rcw_ref.at[s],
                send_sem=send_cw.at[s],
                recv_sem=recv_cw.at[s],
                device_id=(right,),
                device_id_type=pl.DeviceIdType.MESH,
            )
            ccw = pltpu.make_async_remote_copy(
                src_ref=accw_ref.at[cmod(my + s)],
                dst_ref=rccw_ref.at[s],
                send_sem=send_ccw.at[s],
                recv_sem=recv_ccw.at[s],
                device_id=(left,),
                device_id_type=pl.DeviceIdType.MESH,
            )
            cw.start()
            ccw.start()
            return cw, ccw

        def rs_wait_add(s, cw, ccw):
            cw.wait()
            ccw.wait()
            icw = cmod(my - s - 1)
            iccw = cmod(my + s + 1)
            acw_ref[icw] = acw_ref[icw] + rcw_ref[s]
            accw_ref[iccw] = accw_ref[iccw] + rccw_ref[s]

        compute_chunk(my)
        rs0 = rs_start(0)
        compute_chunk(cmod(my - 1))
        compute_chunk(cmod(my + 1))
        rs_wait_add(0, *rs0)
        rs1 = rs_start(1)
        compute_chunk(cmod(my + 2))
        rs_wait_add(1, *rs1)
        rs2 = rs_start(2)
        rs_wait_add(2, *rs2)

        own_cw = cmod(my + 1)
        own_ccw = cmod(my - 1)
        out_ref[pl.ds(own_cw * RC, RC), :CH] = acw_ref[own_cw]
        out_ref[pl.ds(own_ccw * RC, RC), CH:] = accw_ref[own_ccw]

        for s in range(N_DEV - 1):
            c_cw = cmod(my + 1 - s)
            c_ccw = cmod(my - 1 + s)
            src_cw = (acw_ref.at[c_cw] if s == 0
                      else out_ref.at[pl.ds(c_cw * RC, RC), pl.ds(0, CH)])
            src_ccw = (accw_ref.at[c_ccw] if s == 0
                       else out_ref.at[pl.ds(c_ccw * RC, RC), pl.ds(CH, CH)])
            cw = pltpu.make_async_remote_copy(
                src_ref=src_cw,
                dst_ref=out_ref.at[pl.ds(c_cw * RC, RC), pl.ds(0, CH)],
                send_sem=send_cw.at[3 + s],
                recv_sem=recv_cw.at[3 + s],
                device_id=(right,),
                device_id_type=pl.DeviceIdType.MESH,
            )
            ccw = pltpu.make_async_remote_copy(
                src_ref=src_ccw,
                dst_ref=out_ref.at[pl.ds(c_ccw * RC, RC), pl.ds(CH, CH)],
                send_sem=send_ccw.at[3 + s],
                recv_sem=recv_ccw.at[3 + s],
                device_id=(left,),
                device_id_type=pl.DeviceIdType.MESH,
            )
            cw.start()
            ccw.start()
            cw.wait()
            ccw.wait()

    out = pl.pallas_call(
        body,
        out_shape=jax.ShapeDtypeStruct((SQ, D_MODEL), jnp.bfloat16),
        in_specs=[
            pl.BlockSpec(memory_space=pltpu.VMEM),
            pl.BlockSpec(memory_space=pl.ANY),
            pl.BlockSpec(memory_space=pltpu.VMEM),
            pl.BlockSpec(memory_space=pltpu.VMEM),
            pl.BlockSpec(memory_space=pl.ANY),
        ],
        out_specs=pl.BlockSpec(memory_space=pltpu.VMEM),
        scratch_shapes=[
            pltpu.VMEM((SQ, D_LOCAL), jnp.bfloat16),
            pltpu.VMEM((N_DEV, RC, CH), jnp.bfloat16),
            pltpu.VMEM((N_DEV, RC, CH), jnp.bfloat16),
            pltpu.VMEM((N_DEV - 1, RC, CH), jnp.bfloat16),
            pltpu.VMEM((N_DEV - 1, RC, CH), jnp.bfloat16),
            pltpu.VMEM((D_MODEL, D_LOCAL), jnp.float32),
            pltpu.VMEM((D_LOCAL, D_MODEL), jnp.float32),
            pltpu.SemaphoreType.DMA((2,)),
            pltpu.SemaphoreType.DMA((6,)),
            pltpu.SemaphoreType.DMA((6,)),
            pltpu.SemaphoreType.DMA((6,)),
            pltpu.SemaphoreType.DMA((6,)),
        ],
        compiler_params=pltpu.CompilerParams(
            collective_id=0,
            vmem_limit_bytes=100 * 1024 * 1024,
        ),
    )(x2, Wq, k2, v2, Wo)
    return out[None].astype(jnp.float32)


# device time: 68750 ns/iter; 1.3534x vs baseline; 1.0296x over previous
import jax
import jax.numpy as jnp
from jax import lax
from jax.experimental import pallas as pl
from jax.experimental.pallas import tpu as pltpu

N_DEV = 4
SQ = 1024
SKV = 1024
HQ_LOCAL = 8
DH = 128
D_LOCAL = HQ_LOCAL * DH
D_MODEL = 1024
BLK = 64
SCALE = 0.08838834764831843
NEG = -1e9

RC = SQ // N_DEV


def kernel(x, Wq, K_ext, V_ext, Wo):
    x2 = x[0]
    k2 = K_ext.reshape(SKV, HQ_LOCAL * DH)
    v2 = V_ext.reshape(SKV, HQ_LOCAL * DH)

    def body(x_ref, wq_ref, k_ref, v_ref, wo_ref, out_ref,
             ctx_ref, ps_ref, rs_recv_ref, wq_l_ref, wo_l_ref,
             dma_sems, rs_send_sems, rs_recv_sems, ag_send_sems,
             ag_recv_sems):
        my = lax.axis_index("i")

        def cmod(k):
            return lax.rem(k + 4 * N_DEV, N_DEV)

        cp_wq = pltpu.make_async_copy(
            wq_ref.at[:, pl.ds(my * D_LOCAL, D_LOCAL)], wq_l_ref,
            dma_sems.at[0])
        cp_wo = pltpu.make_async_copy(
            wo_ref.at[pl.ds(my * D_LOCAL, D_LOCAL), :], wo_l_ref,
            dma_sems.at[1])
        cp_wq.start()
        cp_wo.start()

        barrier_sem = pltpu.get_barrier_semaphore()
        for d in range(1, N_DEV):
            pl.semaphore_signal(
                barrier_sem, inc=1,
                device_id=(cmod(my + d),),
                device_id_type=pl.DeviceIdType.MESH,
            )
        pl.semaphore_wait(barrier_sem, N_DEV - 1)

        cp_wq.wait()
        cp_wo.wait()
        wq_l = (wq_l_ref[...] * SCALE).astype(jnp.bfloat16)
        wo_l = wo_l_ref[...].astype(jnp.bfloat16)
        xb = x_ref[...].astype(jnp.bfloat16)
        kb = k_ref[...].astype(jnp.bfloat16)
        vb = v_ref[...].astype(jnp.bfloat16)

        di = lax.broadcasted_iota(jnp.int32, (RC, RC), 0) // BLK
        dj = lax.broadcasted_iota(jnp.int32, (RC, RC), 1) // BLK
        diag_mask = dj <= di

        rs_sends = []
        for c in range(N_DEV):
            KL = RC * (c + 1)
            q_c = jnp.dot(xb[c * RC:(c + 1) * RC, :], wq_l,
                          preferred_element_type=jnp.float32
                          ).astype(jnp.bfloat16)
            for h in range(HQ_LOCAL):
                sl = slice(h * DH, (h + 1) * DH)
                s = lax.dot_general(
                    q_c[:, sl], kb[:KL, sl], (((1,), (1,)), ((), ())),
                    preferred_element_type=jnp.float32,
                )
                if c == 0:
                    s = jnp.where(diag_mask, s, NEG)
                else:
                    s = jnp.concatenate(
                        [s[:, :c * RC],
                         jnp.where(diag_mask, s[:, c * RC:], NEG)], axis=1)
                m = jnp.max(s, axis=1, keepdims=True)
                p = jnp.exp(s - m)
                denom = jnp.sum(p, axis=1, keepdims=True)
                ctx = jnp.dot(p.astype(jnp.bfloat16), vb[:KL, sl],
                              preferred_element_type=jnp.float32)
                ctx_ref[c * RC:(c + 1) * RC, sl] = (
                    ctx / denom).astype(jnp.bfloat16)
            pc = jnp.dot(ctx_ref[c * RC:(c + 1) * RC, :], wo_l,
                         preferred_element_type=jnp.float32)
            ps_ref[c] = pc.astype(jnp.bfloat16)

            rdma = pltpu.make_async_remote_copy(
                src_ref=ps_ref.at[c],
                dst_ref=rs_recv_ref.at[cmod(my - c - 1)],
                send_sem=rs_send_sems.at[c],
                recv_sem=rs_recv_sems.at[cmod(my - c - 1)],
                device_id=(c,),
                device_id_type=pl.DeviceIdType.MESH,
            )
            rdma.start()
            rs_sends.append(rdma)

        for j in range(N_DEV):
            pltpu.make_async_remote_copy(
                src_ref=rs_recv_ref.at[j], dst_ref=rs_recv_ref.at[j],
                send_sem=rs_send_sems.at[0], recv_sem=rs_recv_sems.at[j],
                device_id=(my,), device_id_type=pl.DeviceIdType.MESH,
            ).wait_recv()

        red = ((rs_recv_ref[0].astype(jnp.float32)
                + rs_recv_ref[1].astype(jnp.float32))
               + (rs_recv_ref[2].astype(jnp.float32)
                  + rs_recv_ref[3].astype(jnp.float32)))
        out_ref[pl.ds(my * RC, RC), :] = red.astype(jnp.bfloat16)

        ag_sends = []
        for d in range(1, N_DEV):
            rdma = pltpu.make_async_remote_copy(
                src_ref=out_ref.at[pl.ds(my * RC, RC), :],
                dst_ref=out_ref.at[pl.ds(my * RC, RC), :],
                send_sem=ag_send_sems.at[d - 1],
                recv_sem=ag_recv_sems.at[3 - d],
                device_id=(cmod(my + d),),
                device_id_type=pl.DeviceIdType.MESH,
            )
            rdma.start()
            ag_sends.append(rdma)

        for j in range(N_DEV - 1):
            pltpu.make_async_remote_copy(
                src_ref=out_ref.at[pl.ds(my * RC, RC), :],
                dst_ref=out_ref.at[pl.ds(my * RC, RC), :],
                send_sem=ag_send_sems.at[0], recv_sem=ag_recv_sems.at[j],
                device_id=(my,), device_id_type=pl.DeviceIdType.MESH,
            ).wait_recv()

        for rdma in rs_sends:
            rdma.wait_send()
        for rdma in ag_sends:
            rdma.wait_send()

    out = pl.pallas_call(
        body,
        out_shape=jax.ShapeDtypeStruct((SQ, D_MODEL), jnp.bfloat16),
        in_specs=[
            pl.BlockSpec(memory_space=pltpu.MemorySpace.VMEM),
            pl.BlockSpec(memory_space=pl.ANY),
            pl.BlockSpec(memory_space=pltpu.MemorySpace.VMEM),
            pl.BlockSpec(memory_space=pltpu.MemorySpace.VMEM),
            pl.BlockSpec(memory_space=pl.ANY),
        ],
        out_specs=pl.BlockSpec(memory_space=pltpu.MemorySpace.VMEM),
        scratch_shapes=[
            pltpu.VMEM((SQ, D_LOCAL), jnp.bfloat16),
            pltpu.VMEM((N_DEV, RC, D_MODEL), jnp.bfloat16),
            pltpu.VMEM((N_DEV, RC, D_MODEL), jnp.bfloat16),
            pltpu.VMEM((D_MODEL, D_LOCAL), jnp.float32),
            pltpu.VMEM((D_LOCAL, D_MODEL), jnp.float32),
            pltpu.SemaphoreType.DMA((2,)),
            pltpu.SemaphoreType.DMA((N_DEV,)),
            pltpu.SemaphoreType.DMA((N_DEV,)),
            pltpu.SemaphoreType.DMA((N_DEV - 1,)),
            pltpu.SemaphoreType.DMA((N_DEV - 1,)),
        ],
        compiler_params=pltpu.CompilerParams(
            collective_id=0,
            vmem_limit_bytes=100 * 1024 * 1024,
        ),
    )(x2, Wq, k2, v2, Wo)
    return out[None].astype(jnp.float32)
